# baseline (speedup 1.0000x reference)
import jax
import jax.numpy as jnp
from jax import lax
from jax.experimental import pallas as pl
from jax.experimental.pallas import tpu as pltpu
from jax.experimental.pallas import tpu_sc as plsc

B = 32
P = 16800
G = 16
CH = 1680
NCH = P // CH
GRP = CH // 16

NEGPOS_RATIO = 3
VAR0, VAR1 = 0.1, 0.2
MIN_NUM_NEG = 30
THRESHOLD = 0.35
NEG_BIG = -3.4e38
BIG_I = 2 ** 30
LN2 = 0.6931471805599453

_LOG2_C = (
    -8.87469654e-03, 1.21275079e-01, -7.34968305e-01, 2.59924784e+00,
    -5.94110963e+00, 9.20200504e+00, -9.94671090e+00, 8.13017464e+00,
    -3.42103902e+00,
)


def _ln(x):
    b = lax.bitcast_convert_type(x, jnp.int32)
    e = ((b >> 23) - 127).astype(jnp.float32)
    m = lax.bitcast_convert_type((b & 0x7FFFFF) | 0x3F800000, jnp.float32)
    p = jnp.full_like(m, _LOG2_C[0])
    for c in _LOG2_C[1:]:
        p = p * m + c
    return (e + p) * LN2


def _sl1(d):
    a = jnp.abs(d)
    return jnp.where(a < 1.0, 0.5 * a * a, a - 0.5)


def _it16():
    return lax.broadcasted_iota(jnp.int32, (16,), 0)


def _popcnt(mask):
    return plsc.all_reduce_population_count(mask)[0]


def _sc_body(priorsf, confs, clss, locs, lms, tcomp, lmrow, out,
             s_bto, s_bti, s_val, s_stageA, s_stageB, s_tc, s_lm,
             s_tmpf, s_out16):
    wid = lax.axis_index("s") * 2 + lax.axis_index("c")
    fzero = jnp.zeros((16,), jnp.float32)
    izero = jnp.zeros((16,), jnp.int32)
    it16 = _it16()

    def vsum(v):
        for s in (8, 4, 2, 1):
            s_tmpf[...] = v
            v = v + plsc.load_gather(s_tmpf, [it16 ^ s])
        return v[0]

    def prefix_inc(v):
        for s in (1, 2, 4, 8):
            s_tmpf[...] = v
            sh = plsc.load_gather(s_tmpf, [jnp.maximum(it16 - s, 0)])
            v = v + jnp.where(it16 >= s, sh, 0.0)
        return v

    pltpu.sync_copy(tcomp.at[wid], s_tc)
    pltpu.sync_copy(lmrow.at[wid], s_lm)

    gtx1 = s_tc[0]
    gty1 = s_tc[1]
    gtx2 = s_tc[2]
    gty2 = s_tc[3]
    gta = s_tc[4]
    it2 = it16 * 2
    it3 = it16 * 3
    it4 = it16 * 4
    it10 = it16 * 10

    def prior_geom(i):
        b4 = it4 + i * 64
        pcx = plsc.load_gather(s_stageB, [b4])
        pcy = plsc.load_gather(s_stageB, [b4 + 1])
        pw = plsc.load_gather(s_stageB, [b4 + 2])
        ph = plsc.load_gather(s_stageB, [b4 + 3])
        return pcx, pcy, pw, ph

    def p1_chunk(c, carry):
        pltpu.sync_copy(priorsf.at[pl.ds(c * 4 * CH, 4 * CH)],
                        s_stageB.at[pl.ds(0, 4 * CH)])

        def p1_grp(i, carry):
            rb, ri = carry
            o = i * 16
            pcx, pcy, pw, ph = prior_geom(i)
            px1 = pcx - 0.5 * pw
            py1 = pcy - 0.5 * ph
            px2 = pcx + 0.5 * pw
            py2 = pcy + 0.5 * ph
            pa = pw * ph
            pidx = c * CH + o + it16
            best_o = fzero
            best_i = izero
            rb2 = []
            ri2 = []
            for g in range(G):
                iw = jnp.maximum(
                    jnp.minimum(gtx2[g], px2) - jnp.maximum(gtx1[g], px1), 0.0)
                ih = jnp.maximum(
                    jnp.minimum(gty2[g], py2) - jnp.maximum(gty1[g], py1), 0.0)
                inter = iw * ih
                iou = inter / ((gta[g] + pa) - inter)
                upd = iou > best_o
                best_o = jnp.where(upd, iou, best_o)
                best_i = jnp.where(upd, g, best_i)
                gupd = iou > rb[g]
                rb2.append(jnp.where(gupd, iou, rb[g]))
                ri2.append(jnp.where(gupd, pidx, ri[g]))

            s_bto[pl.ds(c * CH + o, 16)] = best_o
            s_bti[pl.ds(c * CH + o, 16)] = best_i
            return tuple(rb2), tuple(ri2)

        return lax.fori_loop(0, GRP, p1_grp, carry)

    rb, ri = lax.fori_loop(0, NCH, p1_chunk,
                           ((fzero,) * G, (izero,) * G))

    gb = []
    gbi = []
    for g in range(G):
        sk, _ = plsc.sort_key_val(rb[g], ri[g], descending=True)
        m = sk[0]
        cand = jnp.where(rb[g] == m, ri[g], jnp.int32(BIG_I))
        sk2, _ = plsc.sort_key_val(cand, cand)
        gb.append(m)
        gbi.append(sk2[0])

    lane0 = it16 == 0
    for g in range(G):
        jv = izero + gbi[g]
        fill = jnp.where(gb[g] >= 0.2, 2.0, NEG_BIG)
        old = plsc.load_gather(s_bto, [jv])
        plsc.store_scatter(s_bto, [jv], jnp.maximum(old, fill), mask=lane0)
        plsc.store_scatter(s_bti, [jv], izero + g, mask=lane0)

    row5 = jnp.full((16,), 5, jnp.int32)

    def p2_chunk(c, carry):
        pltpu.sync_copy(confs.at[wid, pl.ds(c * 2 * CH, 2 * CH)],
                        s_stageA.at[pl.ds(0, 2 * CH)])

        def p2_grp(i, carry):
            acc_posce, acc_npos = carry
            o = i * 16
            sl = pl.ds(c * CH + o, 16)
            b2 = it2 + i * 32
            c0 = plsc.load_gather(s_stageA, [b2])
            c1 = plsc.load_gather(s_stageA, [b2 + 1])
            bto = s_bto[sl]
            bti = s_bti[sl]
            lbl = plsc.load_gather(s_tc, [row5, bti])
            ct = jnp.where(bto < THRESHOLD, 0.0, lbl)
            s_bto[sl] = ct
            pos = ct != 0.0
            mx = jnp.maximum(c0, c1)
            z = jnp.exp(-jnp.abs(c0 - c1))
            lse = mx + _ln(1.0 + z)
            ce = lse - jnp.where(pos, c1, c0)
            val = jnp.where(pos, 0.0, ce)
            s_val[sl] = val
            return (acc_posce + jnp.where(pos, ce, 0.0),
                    acc_npos + jnp.where(pos, 1.0, 0.0))

        return lax.fori_loop(0, GRP, p2_grp, carry)

    acc_posce, acc_npos = lax.fori_loop(0, NCH, p2_chunk, (fzero, fzero))
    sum_posce = vsum(acc_posce)
    npos_f = vsum(acc_npos)
    nn_f = jnp.minimum(3.0 * npos_f, float(P - 1))
    nn_f = jnp.maximum(nn_f, float(MIN_NUM_NEG))

    def count_ge(t):
        def cg(i, acc):
            kv = lax.bitcast_convert_type(s_val[pl.ds(i * 16, 16)], jnp.int32)
            return acc + jnp.where(kv >= t, 1.0, 0.0)

        acc = lax.fori_loop(0, P // 16, cg, fzero, unroll=4)
        return vsum(acc)

    def bit_iter(b, v):
        cand = v | lax.shift_left(jnp.int32(1), 30 - b)
        return jnp.where(count_ge(cand) >= nn_f, cand, v)

    vthr = lax.fori_loop(0, 31, bit_iter, jnp.int32(0))
    c_gt_f = count_ge(vthr + 1)
    t_take = nn_f - c_gt_f

    def p4_chunk(c, carry):
        pltpu.sync_copy(clss.at[wid, pl.ds(c * 3 * CH, 3 * CH)],
                        s_stageA.at[pl.ds(0, 3 * CH)])

        def p4_grp(i, carry):
            run, acc_negval, acc_r = carry
            o = i * 16
            sl = pl.ds(c * CH + o, 16)
            valv = s_val[sl]
            kv = lax.bitcast_convert_type(valv, jnp.int32)
            ct = s_bto[sl]
            pos = ct != 0.0
            eq = kv == vthr
            n_eq = _popcnt(eq)
            gt = kv > vthr

            def with_ties():
                cs = prefix_inc(jnp.where(eq, 1.0, 0.0))
                return gt | (eq & ((run + cs) <= t_take))

            taken = lax.cond(n_eq > 0, with_ties, lambda: gt)
            run = run + n_eq.astype(jnp.float32)
            acc_negval = acc_negval + jnp.where(taken, valv, 0.0)
            selm = pos | taken

            def with_ce():
                b3 = it3 + i * 48
                x0 = plsc.load_gather(s_stageA, [b3])
                x1 = plsc.load_gather(s_stageA, [b3 + 1])
                x2 = plsc.load_gather(s_stageA, [b3 + 2])
                m3 = jnp.maximum(x0, jnp.maximum(x1, x2))
                s3 = jnp.exp(x0 - m3) + jnp.exp(x1 - m3) + jnp.exp(x2 - m3)
                lse3 = m3 + _ln(s3)
                logit = jnp.where(ct == 1.0, x1, jnp.where(ct == 2.0, x2, x0))
                return acc_r + jnp.where(selm, lse3 - logit, 0.0)

            acc_r = lax.cond(_popcnt(selm) > 0, with_ce, lambda: acc_r)
            return run, acc_negval, acc_r

        return lax.fori_loop(0, GRP, p4_grp, carry)

    _, acc_negval, acc_r = lax.fori_loop(
        0, NCH, p4_chunk, (jnp.float32(0.0), fzero, fzero))
    sum_negval = vsum(acc_negval)
    sum_r = vsum(acc_r)

    z16 = izero

    def p5_chunk(c, acc_l):
        pltpu.sync_copy(locs.at[wid, pl.ds(c * 4 * CH, 4 * CH)],
                        s_stageA.at[pl.ds(0, 4 * CH)])
        pltpu.sync_copy(priorsf.at[pl.ds(c * 4 * CH, 4 * CH)],
                        s_stageB.at[pl.ds(0, 4 * CH)])

        def p5_grp(i, acc_l):
            o = i * 16
            sl = pl.ds(c * CH + o, 16)
            ct = s_bto[sl]
            pos = ct != 0.0

            def with_loc():
                pcx, pcy, pw, ph = prior_geom(i)
                bti = s_bti[sl]
                tx1 = plsc.load_gather(s_tc, [z16, bti])
                ty1 = plsc.load_gather(s_tc, [z16 + 1, bti])
                tx2 = plsc.load_gather(s_tc, [z16 + 2, bti])
                ty2 = plsc.load_gather(s_tc, [z16 + 3, bti])
                gx = (0.5 * (tx1 + tx2) - pcx) / (VAR0 * pw)
                gy = (0.5 * (ty1 + ty2) - pcy) / (VAR0 * ph)
                gw = _ln((tx2 - tx1) / pw) * (1.0 / VAR1)
                gh = _ln((ty2 - ty1) / ph) * (1.0 / VAR1)
                b4 = it4 + i * 64
                l0 = plsc.load_gather(s_stageA, [b4])
                l1 = plsc.load_gather(s_stageA, [b4 + 1])
                l2 = plsc.load_gather(s_stageA, [b4 + 2])
                l3 = plsc.load_gather(s_stageA, [b4 + 3])
                tot = (_sl1(l0 - gx) + _sl1(l1 - gy)
                       + _sl1(l2 - gw) + _sl1(l3 - gh))
                return acc_l + jnp.where(pos, tot, 0.0)

            return lax.cond(_popcnt(pos) > 0, with_loc, lambda: acc_l)

        return lax.fori_loop(0, GRP, p5_grp, acc_l)

    acc_l = lax.fori_loop(0, NCH, p5_chunk, fzero)
    sum_l = vsum(acc_l)

    def p6_chunk(c, carry):
        pltpu.sync_copy(lms.at[wid, pl.ds(c * 10 * CH, 10 * CH)],
                        s_stageA.at[pl.ds(0, 10 * CH)])
        pltpu.sync_copy(priorsf.at[pl.ds(c * 4 * CH, 4 * CH)],
                        s_stageB.at[pl.ds(0, 4 * CH)])

        def p6_grp(i, carry):
            acc_lm, acc_nplm = carry
            o = i * 16
            sl = pl.ds(c * CH + o, 16)
            ct = s_bto[sl]
            bti = s_bti[sl]
            mlm0 = plsc.load_gather(s_lm, [bti, z16])
            pos1 = (ct == 1.0) & (mlm0 >= 0.0)

            def with_lm():
                pcx, pcy, pw, ph = prior_geom(i)
                b10 = it10 + i * 160
                tot = fzero
                for cc in range(10):
                    lmv = plsc.load_gather(s_stageA, [b10 + cc])
                    mlmc = plsc.load_gather(s_lm, [bti, z16 + cc])
                    pc = pcx if cc % 2 == 0 else pcy
                    pww = pw if cc % 2 == 0 else ph
                    t = (mlmc - pc) / (VAR0 * pww)
                    tot = tot + _sl1(lmv - t)
                return (acc_lm + jnp.where(pos1, tot, 0.0),
                        acc_nplm + jnp.where(pos1, 1.0, 0.0))

            return lax.cond(_popcnt(pos1) > 0, with_lm,
                            lambda: (acc_lm, acc_nplm))

        return lax.fori_loop(0, GRP, p6_grp, carry)

    acc_lm, acc_nplm = lax.fori_loop(0, NCH, p6_chunk, (fzero, fzero))
    sum_lm = vsum(acc_lm)
    nplm_f = vsum(acc_nplm)

    outv = fzero
    vals = (
        sum_l,
        sum_r,
        sum_posce + sum_negval,
        sum_lm,
        npos_f,
        nn_f,
        nplm_f,
    )
    for k, sv in enumerate(vals):
        outv = jnp.where(it16 == k, sv, outv)
    s_out16[...] = outv
    pltpu.sync_copy(s_out16, out.at[wid])


def _fin_body(x_ref, o_ref):
    x = x_ref[...]
    s = jnp.sum(x, axis=0, keepdims=True)
    col = lax.broadcasted_iota(jnp.int32, (1, 16), 1)

    def pick(k):
        return jnp.sum(jnp.where(col == k, s, 0.0))

    N = jnp.maximum(pick(4), 1.0)
    N_neg = jnp.maximum(pick(5), 1.0)
    N1 = jnp.maximum(pick(6), 1.0)
    loss_l = pick(0) / N
    loss_r = pick(1) / (N + N_neg) * (NEGPOS_RATIO + 1)
    loss_c = pick(2) / (N + N_neg) * (NEGPOS_RATIO + 1)
    loss_lm = pick(3) / N1
    o = jnp.where(col == 0, loss_l, 0.0)
    o = jnp.where(col == 1, loss_r, o)
    o = jnp.where(col == 2, loss_c, o)
    o = jnp.where(col == 3, loss_lm, o)
    o_ref[...] = o


def kernel(loc_data, class_data, conf_data, landm_data, priors, targets):
    f32 = jnp.float32
    priorsf = priors.reshape(P * 4)
    confs = conf_data.reshape(B, P * 2)
    clss = class_data.reshape(B, P * 3)
    locs = loc_data.reshape(B, P * 4)
    lms = landm_data.reshape(B, P * 10)
    tr = targets[:, :, :4]
    tarea = (tr[:, :, 2] - tr[:, :, 0]) * (tr[:, :, 3] - tr[:, :, 1])
    tcomp = jnp.stack([tr[:, :, 0], tr[:, :, 1], tr[:, :, 2], tr[:, :, 3],
                       tarea, targets[:, :, 14]], 1)
    lmrow = targets[:, :, 4:14]

    mesh = plsc.VectorSubcoreMesh(core_axis_name="c", subcore_axis_name="s")
    sc = pl.kernel(
        _sc_body,
        mesh=mesh,
        compiler_params=pltpu.CompilerParams(
            needs_layout_passes=False, use_tc_tiling_on_sc=False),
        out_type=jax.ShapeDtypeStruct((B, 16), f32),
        scratch_types=[
            pltpu.VMEM((P,), f32),
            pltpu.VMEM((P,), jnp.int32),
            pltpu.VMEM((P,), f32),
            pltpu.VMEM((10 * CH,), f32),
            pltpu.VMEM((4 * CH,), f32),
            pltpu.VMEM((6, G), f32),
            pltpu.VMEM((G, 10), f32),
            pltpu.VMEM((16,), f32),
            pltpu.VMEM((16,), f32),
        ],
    )
    partials = sc(priorsf, confs, clss, locs, lms, tcomp, lmrow)

    fin = pl.pallas_call(
        _fin_body,
        out_shape=jax.ShapeDtypeStruct((1, 16), f32),
    )(partials)
    return (fin[0, 0], fin[0, 1], fin[0, 2], fin[0, 3])


# speedup vs baseline: 1.3430x; 1.3430x over previous
import jax
import jax.numpy as jnp
from jax import lax
from jax.experimental import pallas as pl
from jax.experimental.pallas import tpu as pltpu
from jax.experimental.pallas import tpu_sc as plsc

B = 32
P = 16800
G = 16
CH = 1680
NCH = P // CH
GRP = CH // 16

NEGPOS_RATIO = 3
VAR0, VAR1 = 0.1, 0.2
MIN_NUM_NEG = 30
THRESHOLD = 0.35
NEG_BIG = -3.4e38
BIG_I = 2 ** 30
LN2 = 0.6931471805599453

_LOG2_C = (
    -8.87469654e-03, 1.21275079e-01, -7.34968305e-01, 2.59924784e+00,
    -5.94110963e+00, 9.20200504e+00, -9.94671090e+00, 8.13017464e+00,
    -3.42103902e+00,
)


def _ln(x):
    b = lax.bitcast_convert_type(x, jnp.int32)
    e = ((b >> 23) - 127).astype(jnp.float32)
    m = lax.bitcast_convert_type((b & 0x7FFFFF) | 0x3F800000, jnp.float32)
    p = jnp.full_like(m, _LOG2_C[0])
    for c in _LOG2_C[1:]:
        p = p * m + c
    return (e + p) * LN2


def _sl1(d):
    a = jnp.abs(d)
    return jnp.where(a < 1.0, 0.5 * a * a, a - 0.5)


def _it16():
    return lax.broadcasted_iota(jnp.int32, (16,), 0)


def _popcnt(mask):
    return plsc.all_reduce_population_count(mask)[0]


def _sc_body(psoa, confs, clss, locs, lms, tcomp, lmrow, out,
             s_bto, s_bti, s_val, s_stageA, s_stageB, s_tc, s_lm,
             s_tmpf, s_out16):
    wid = lax.axis_index("s") * 2 + lax.axis_index("c")
    fzero = jnp.zeros((16,), jnp.float32)
    izero = jnp.zeros((16,), jnp.int32)
    it16 = _it16()

    def vsum(v):
        for s in (8, 4, 2, 1):
            s_tmpf[...] = v
            v = v + plsc.load_gather(s_tmpf, [it16 ^ s])
        return v[0]

    def prefix_inc(v):
        for s in (1, 2, 4, 8):
            s_tmpf[...] = v
            sh = plsc.load_gather(s_tmpf, [jnp.maximum(it16 - s, 0)])
            v = v + jnp.where(it16 >= s, sh, 0.0)
        return v

    pltpu.sync_copy(tcomp.at[wid], s_tc)
    pltpu.sync_copy(lmrow.at[wid], s_lm)

    gtx1 = s_tc[0]
    gty1 = s_tc[1]
    gtx2 = s_tc[2]
    gty2 = s_tc[3]
    gta = s_tc[4]

    def p1_chunk(c, carry):
        pltpu.sync_copy(psoa.at[c], s_stageB.at[pl.ds(0, 5 * CH)])

        def p1_grp(i, carry):
            rb, ri = carry
            o = i * 16
            px1 = s_stageB[pl.ds(o, 16)]
            py1 = s_stageB[pl.ds(CH + o, 16)]
            px2 = s_stageB[pl.ds(2 * CH + o, 16)]
            py2 = s_stageB[pl.ds(3 * CH + o, 16)]
            pa = s_stageB[pl.ds(4 * CH + o, 16)]
            pidx = c * CH + o + it16
            best_o = fzero
            best_i = izero
            rb2 = []
            ri2 = []
            for g in range(G):
                iw = jnp.maximum(
                    jnp.minimum(gtx2[g], px2) - jnp.maximum(gtx1[g], px1), 0.0)
                ih = jnp.maximum(
                    jnp.minimum(gty2[g], py2) - jnp.maximum(gty1[g], py1), 0.0)
                inter = iw * ih
                iou = inter / ((gta[g] + pa) - inter)
                upd = iou > best_o
                best_o = jnp.where(upd, iou, best_o)
                best_i = jnp.where(upd, g, best_i)
                gupd = iou > rb[g]
                rb2.append(jnp.where(gupd, iou, rb[g]))
                ri2.append(jnp.where(gupd, pidx, ri[g]))

            s_bto[pl.ds(c * CH + o, 16)] = best_o
            s_bti[pl.ds(c * CH + o, 16)] = best_i
            return tuple(rb2), tuple(ri2)

        return lax.fori_loop(0, GRP, p1_grp, carry, unroll=3)

    rb, ri = lax.fori_loop(0, NCH, p1_chunk,
                           ((fzero,) * G, (izero,) * G))

    gb = []
    gbi = []
    for g in range(G):
        sk, _ = plsc.sort_key_val(rb[g], ri[g], descending=True)
        m = sk[0]
        cand = jnp.where(rb[g] == m, ri[g], jnp.int32(BIG_I))
        sk2, _ = plsc.sort_key_val(cand, cand)
        gb.append(m)
        gbi.append(sk2[0])

    lane0 = it16 == 0
    for g in range(G):
        jv = izero + gbi[g]
        fill = jnp.where(gb[g] >= 0.2, 2.0, NEG_BIG)
        old = plsc.load_gather(s_bto, [jv])
        plsc.store_scatter(s_bto, [jv], jnp.maximum(old, fill), mask=lane0)
        plsc.store_scatter(s_bti, [jv], izero + g, mask=lane0)

    row5 = jnp.full((16,), 5, jnp.int32)

    def p2_chunk(c, carry):
        pltpu.sync_copy(confs.at[wid, c], s_stageA.at[pl.ds(0, 2 * CH)])

        def p2_grp(i, carry):
            acc_posce, acc_npos = carry
            o = i * 16
            sl = pl.ds(c * CH + o, 16)
            c0 = s_stageA[pl.ds(o, 16)]
            c1 = s_stageA[pl.ds(CH + o, 16)]
            bto = s_bto[sl]
            bti = s_bti[sl]
            lbl = plsc.load_gather(s_tc, [row5, bti])
            ct = jnp.where(bto < THRESHOLD, 0.0, lbl)
            s_bto[sl] = ct
            pos = ct != 0.0
            mx = jnp.maximum(c0, c1)
            z = jnp.exp(-jnp.abs(c0 - c1))
            lse = mx + _ln(1.0 + z)
            ce = lse - jnp.where(pos, c1, c0)
            val = jnp.where(pos, 0.0, ce)
            s_val[sl] = val
            return (acc_posce + jnp.where(pos, ce, 0.0),
                    acc_npos + jnp.where(pos, 1.0, 0.0))

        return lax.fori_loop(0, GRP, p2_grp, carry, unroll=3)

    acc_posce, acc_npos = lax.fori_loop(0, NCH, p2_chunk, (fzero, fzero))
    sum_posce = vsum(acc_posce)
    npos_f = vsum(acc_npos)
    nn_f = jnp.minimum(3.0 * npos_f, float(P - 1))
    nn_f = jnp.maximum(nn_f, float(MIN_NUM_NEG))

    def count_ge(t):
        def cg(i, acc):
            kv = lax.bitcast_convert_type(s_val[pl.ds(i * 16, 16)], jnp.int32)
            return acc + jnp.where(kv >= t, 1.0, 0.0)

        acc = lax.fori_loop(0, P // 16, cg, fzero, unroll=4)
        return vsum(acc)

    def bit_iter(b, v):
        cand = v | lax.shift_left(jnp.int32(1), 30 - b)
        return jnp.where(count_ge(cand) >= nn_f, cand, v)

    vthr = lax.fori_loop(0, 31, bit_iter, jnp.int32(0))
    c_gt_f = count_ge(vthr + 1)
    t_take = nn_f - c_gt_f

    def p4_chunk(c, carry):
        pltpu.sync_copy(clss.at[wid, c], s_stageA.at[pl.ds(0, 3 * CH)])

        def p4_grp(i, carry):
            run, acc_negval, acc_r = carry
            o = i * 16
            sl = pl.ds(c * CH + o, 16)
            valv = s_val[sl]
            kv = lax.bitcast_convert_type(valv, jnp.int32)
            ct = s_bto[sl]
            pos = ct != 0.0
            eq = kv == vthr
            n_eq = _popcnt(eq)
            gt = kv > vthr

            def with_ties():
                cs = prefix_inc(jnp.where(eq, 1.0, 0.0))
                return gt | (eq & ((run + cs) <= t_take))

            taken = lax.cond(n_eq > 0, with_ties, lambda: gt)
            run = run + n_eq.astype(jnp.float32)
            acc_negval = acc_negval + jnp.where(taken, valv, 0.0)
            selm = pos | taken

            def with_ce():
                x0 = s_stageA[pl.ds(o, 16)]
                x1 = s_stageA[pl.ds(CH + o, 16)]
                x2 = s_stageA[pl.ds(2 * CH + o, 16)]
                m3 = jnp.maximum(x0, jnp.maximum(x1, x2))
                s3 = jnp.exp(x0 - m3) + jnp.exp(x1 - m3) + jnp.exp(x2 - m3)
                lse3 = m3 + _ln(s3)
                logit = jnp.where(ct == 1.0, x1, jnp.where(ct == 2.0, x2, x0))
                return acc_r + jnp.where(selm, lse3 - logit, 0.0)

            acc_r = lax.cond(_popcnt(selm) > 0, with_ce, lambda: acc_r)
            return run, acc_negval, acc_r

        return lax.fori_loop(0, GRP, p4_grp, carry, unroll=3)

    _, acc_negval, acc_r = lax.fori_loop(
        0, NCH, p4_chunk, (jnp.float32(0.0), fzero, fzero))
    sum_negval = vsum(acc_negval)
    sum_r = vsum(acc_r)

    z16 = izero

    def p5_chunk(c, acc_l):
        pltpu.sync_copy(locs.at[wid, c], s_stageA.at[pl.ds(0, 4 * CH)])
        pltpu.sync_copy(psoa.at[c], s_stageB.at[pl.ds(0, 5 * CH)])

        def p5_grp(i, acc_l):
            o = i * 16
            sl = pl.ds(c * CH + o, 16)
            ct = s_bto[sl]
            pos = ct != 0.0

            def with_loc():
                px1 = s_stageB[pl.ds(o, 16)]
                py1 = s_stageB[pl.ds(CH + o, 16)]
                px2 = s_stageB[pl.ds(2 * CH + o, 16)]
                py2 = s_stageB[pl.ds(3 * CH + o, 16)]
                pcx = 0.5 * (px1 + px2)
                pcy = 0.5 * (py1 + py2)
                pw = px2 - px1
                ph = py2 - py1
                bti = s_bti[sl]
                tx1 = plsc.load_gather(s_tc, [z16, bti])
                ty1 = plsc.load_gather(s_tc, [z16 + 1, bti])
                tx2 = plsc.load_gather(s_tc, [z16 + 2, bti])
                ty2 = plsc.load_gather(s_tc, [z16 + 3, bti])
                gx = (0.5 * (tx1 + tx2) - pcx) / (VAR0 * pw)
                gy = (0.5 * (ty1 + ty2) - pcy) / (VAR0 * ph)
                gw = _ln((tx2 - tx1) / pw) * (1.0 / VAR1)
                gh = _ln((ty2 - ty1) / ph) * (1.0 / VAR1)
                l0 = s_stageA[pl.ds(o, 16)]
                l1 = s_stageA[pl.ds(CH + o, 16)]
                l2 = s_stageA[pl.ds(2 * CH + o, 16)]
                l3 = s_stageA[pl.ds(3 * CH + o, 16)]
                tot = (_sl1(l0 - gx) + _sl1(l1 - gy)
                       + _sl1(l2 - gw) + _sl1(l3 - gh))
                return acc_l + jnp.where(pos, tot, 0.0)

            return lax.cond(_popcnt(pos) > 0, with_loc, lambda: acc_l)

        return lax.fori_loop(0, GRP, p5_grp, acc_l, unroll=3)

    acc_l = lax.fori_loop(0, NCH, p5_chunk, fzero)
    sum_l = vsum(acc_l)

    def p6_chunk(c, carry):
        pltpu.sync_copy(lms.at[wid, c], s_stageA.at[pl.ds(0, 10 * CH)])
        pltpu.sync_copy(psoa.at[c], s_stageB.at[pl.ds(0, 5 * CH)])

        def p6_grp(i, carry):
            acc_lm, acc_nplm = carry
            o = i * 16
            sl = pl.ds(c * CH + o, 16)
            ct = s_bto[sl]
            bti = s_bti[sl]
            mlm0 = plsc.load_gather(s_lm, [bti, z16])
            pos1 = (ct == 1.0) & (mlm0 >= 0.0)

            def with_lm():
                px1 = s_stageB[pl.ds(o, 16)]
                py1 = s_stageB[pl.ds(CH + o, 16)]
                px2 = s_stageB[pl.ds(2 * CH + o, 16)]
                py2 = s_stageB[pl.ds(3 * CH + o, 16)]
                pcx = 0.5 * (px1 + px2)
                pcy = 0.5 * (py1 + py2)
                pw = px2 - px1
                ph = py2 - py1
                tot = fzero
                for cc in range(10):
                    lmv = s_stageA[pl.ds(cc * CH + o, 16)]
                    mlmc = plsc.load_gather(s_lm, [bti, z16 + cc])
                    pc = pcx if cc % 2 == 0 else pcy
                    pww = pw if cc % 2 == 0 else ph
                    t = (mlmc - pc) / (VAR0 * pww)
                    tot = tot + _sl1(lmv - t)
                return (acc_lm + jnp.where(pos1, tot, 0.0),
                        acc_nplm + jnp.where(pos1, 1.0, 0.0))

            return lax.cond(_popcnt(pos1) > 0, with_lm,
                            lambda: (acc_lm, acc_nplm))

        return lax.fori_loop(0, GRP, p6_grp, carry, unroll=3)

    acc_lm, acc_nplm = lax.fori_loop(0, NCH, p6_chunk, (fzero, fzero))
    sum_lm = vsum(acc_lm)
    nplm_f = vsum(acc_nplm)

    outv = fzero
    vals = (
        sum_l,
        sum_r,
        sum_posce + sum_negval,
        sum_lm,
        npos_f,
        nn_f,
        nplm_f,
    )
    for k, sv in enumerate(vals):
        outv = jnp.where(it16 == k, sv, outv)
    s_out16[...] = outv
    pltpu.sync_copy(s_out16, out.at[wid])


def _fin_body(x_ref, o_ref):
    x = x_ref[...]
    s = jnp.sum(x, axis=0, keepdims=True)
    col = lax.broadcasted_iota(jnp.int32, (1, 16), 1)

    def pick(k):
        return jnp.sum(jnp.where(col == k, s, 0.0))

    N = jnp.maximum(pick(4), 1.0)
    N_neg = jnp.maximum(pick(5), 1.0)
    N1 = jnp.maximum(pick(6), 1.0)
    loss_l = pick(0) / N
    loss_r = pick(1) / (N + N_neg) * (NEGPOS_RATIO + 1)
    loss_c = pick(2) / (N + N_neg) * (NEGPOS_RATIO + 1)
    loss_lm = pick(3) / N1
    o = jnp.where(col == 0, loss_l, 0.0)
    o = jnp.where(col == 1, loss_r, o)
    o = jnp.where(col == 2, loss_c, o)
    o = jnp.where(col == 3, loss_lm, o)
    o_ref[...] = o


def kernel(loc_data, class_data, conf_data, landm_data, priors, targets):
    f32 = jnp.float32
    pf1 = priors[:, :2] - priors[:, 2:] / 2.0
    pf2 = priors[:, :2] + priors[:, 2:] / 2.0
    px1, py1 = pf1[:, 0], pf1[:, 1]
    px2, py2 = pf2[:, 0], pf2[:, 1]
    parea = (px2 - px1) * (py2 - py1)
    psoa = (jnp.stack([px1, py1, px2, py2, parea], 0)
            .reshape(5, NCH, CH).transpose(1, 0, 2).reshape(NCH, 5 * CH))

    def soa(x, c):
        return (x.transpose(0, 2, 1).reshape(B, c, NCH, CH)
                .transpose(0, 2, 1, 3).reshape(B, NCH, c * CH))

    confs = soa(conf_data, 2)
    clss = soa(class_data, 3)
    locs = soa(loc_data, 4)
    lms = soa(landm_data, 10)
    tr = targets[:, :, :4]
    tarea = (tr[:, :, 2] - tr[:, :, 0]) * (tr[:, :, 3] - tr[:, :, 1])
    tcomp = jnp.stack([tr[:, :, 0], tr[:, :, 1], tr[:, :, 2], tr[:, :, 3],
                       tarea, targets[:, :, 14]], 1)
    lmrow = targets[:, :, 4:14]

    mesh = plsc.VectorSubcoreMesh(core_axis_name="c", subcore_axis_name="s")
    sc = pl.kernel(
        _sc_body,
        mesh=mesh,
        compiler_params=pltpu.CompilerParams(
            needs_layout_passes=False, use_tc_tiling_on_sc=False),
        out_type=jax.ShapeDtypeStruct((B, 16), f32),
        scratch_types=[
            pltpu.VMEM((P,), f32),
            pltpu.VMEM((P,), jnp.int32),
            pltpu.VMEM((P,), f32),
            pltpu.VMEM((10 * CH,), f32),
            pltpu.VMEM((5 * CH,), f32),
            pltpu.VMEM((6, G), f32),
            pltpu.VMEM((G, 10), f32),
            pltpu.VMEM((16,), f32),
            pltpu.VMEM((16,), f32),
        ],
    )
    partials = sc(psoa, confs, clss, locs, lms, tcomp, lmrow)

    fin = pl.pallas_call(
        _fin_body,
        out_shape=jax.ShapeDtypeStruct((1, 16), f32),
    )(partials)
    return (fin[0, 0], fin[0, 1], fin[0, 2], fin[0, 3])


# speedup vs baseline: 1.4829x; 1.1042x over previous
import jax
import jax.numpy as jnp
from jax import lax
from jax.experimental import pallas as pl
from jax.experimental.pallas import tpu as pltpu
from jax.experimental.pallas import tpu_sc as plsc

B = 32
P = 16800
G = 16
CH = 3360
NCH = P // CH
GRP = CH // 16

NEGPOS_RATIO = 3
VAR0, VAR1 = 0.1, 0.2
MIN_NUM_NEG = 30
THRESHOLD = 0.35
NEG_BIG = -3.4e38
BIG_I = 2 ** 30
LN2 = 0.6931471805599453

_LOG2_C = (
    -8.87469654e-03, 1.21275079e-01, -7.34968305e-01, 2.59924784e+00,
    -5.94110963e+00, 9.20200504e+00, -9.94671090e+00, 8.13017464e+00,
    -3.42103902e+00,
)


def _ln(x):
    b = lax.bitcast_convert_type(x, jnp.int32)
    e = ((b >> 23) - 127).astype(jnp.float32)
    m = lax.bitcast_convert_type((b & 0x7FFFFF) | 0x3F800000, jnp.float32)
    p = jnp.full_like(m, _LOG2_C[0])
    for c in _LOG2_C[1:]:
        p = p * m + c
    return (e + p) * LN2


def _sl1(d):
    a = jnp.abs(d)
    return jnp.where(a < 1.0, 0.5 * a * a, a - 0.5)


def _it16():
    return lax.broadcasted_iota(jnp.int32, (16,), 0)


def _popcnt(mask):
    return plsc.all_reduce_population_count(mask)[0]


def _sc_body(psoa, confs, clss, locs, lms, tcomp, lmrow, out,
             s_bto, s_bti, s_val, s_stageA, s_stageB, s_tc, s_lm,
             s_tmpf, s_out16):
    wid = lax.axis_index("s") * 2 + lax.axis_index("c")
    fzero = jnp.zeros((16,), jnp.float32)
    izero = jnp.zeros((16,), jnp.int32)
    it16 = _it16()

    def vsum(v):
        for s in (8, 4, 2, 1):
            s_tmpf[...] = v
            v = v + plsc.load_gather(s_tmpf, [it16 ^ s])
        return v[0]

    def prefix_inc(v):
        for s in (1, 2, 4, 8):
            s_tmpf[...] = v
            sh = plsc.load_gather(s_tmpf, [jnp.maximum(it16 - s, 0)])
            v = v + jnp.where(it16 >= s, sh, 0.0)
        return v

    pltpu.sync_copy(tcomp.at[wid], s_tc)
    pltpu.sync_copy(lmrow.at[wid], s_lm)

    gtx1 = s_tc[0]
    gty1 = s_tc[1]
    gtx2 = s_tc[2]
    gty2 = s_tc[3]
    gta = s_tc[4]

    def p1_chunk(c, carry):
        pltpu.sync_copy(psoa.at[c], s_stageB.at[pl.ds(0, 5 * CH)])

        def p1_grp(i, carry):
            rb, ri = carry
            o = i * 16
            px1 = s_stageB[pl.ds(o, 16)]
            py1 = s_stageB[pl.ds(CH + o, 16)]
            px2 = s_stageB[pl.ds(2 * CH + o, 16)]
            py2 = s_stageB[pl.ds(3 * CH + o, 16)]
            pa = s_stageB[pl.ds(4 * CH + o, 16)]
            pidx = c * CH + o + it16
            best_o = fzero
            best_i = izero
            rb2 = []
            ri2 = []
            for g in range(G):
                iw = jnp.maximum(
                    jnp.minimum(gtx2[g], px2) - jnp.maximum(gtx1[g], px1), 0.0)
                ih = jnp.maximum(
                    jnp.minimum(gty2[g], py2) - jnp.maximum(gty1[g], py1), 0.0)
                inter = iw * ih
                iou = inter / ((gta[g] + pa) - inter)
                upd = iou > best_o
                best_o = jnp.where(upd, iou, best_o)
                best_i = jnp.where(upd, g, best_i)
                gupd = iou > rb[g]
                rb2.append(jnp.where(gupd, iou, rb[g]))
                ri2.append(jnp.where(gupd, pidx, ri[g]))

            s_bto[pl.ds(c * CH + o, 16)] = best_o
            s_bti[pl.ds(c * CH + o, 16)] = best_i
            return tuple(rb2), tuple(ri2)

        return lax.fori_loop(0, GRP, p1_grp, carry, unroll=3)

    rb, ri = lax.fori_loop(0, NCH, p1_chunk,
                           ((fzero,) * G, (izero,) * G))

    gb = []
    gbi = []
    for g in range(G):
        sk, _ = plsc.sort_key_val(rb[g], ri[g], descending=True)
        m = sk[0]
        cand = jnp.where(rb[g] == m, ri[g], jnp.int32(BIG_I))
        sk2, _ = plsc.sort_key_val(cand, cand)
        gb.append(m)
        gbi.append(sk2[0])

    lane0 = it16 == 0
    for g in range(G):
        jv = izero + gbi[g]
        fill = jnp.where(gb[g] >= 0.2, 2.0, NEG_BIG)
        old = plsc.load_gather(s_bto, [jv])
        plsc.store_scatter(s_bto, [jv], jnp.maximum(old, fill), mask=lane0)
        plsc.store_scatter(s_bti, [jv], izero + g, mask=lane0)

    row5 = jnp.full((16,), 5, jnp.int32)

    def p2_chunk(c, carry):
        pltpu.sync_copy(confs.at[wid, c], s_stageA.at[pl.ds(0, 2 * CH)])

        def p2_grp(i, carry):
            acc_posce, acc_npos = carry
            o = i * 16
            sl = pl.ds(c * CH + o, 16)
            c0 = s_stageA[pl.ds(o, 16)]
            c1 = s_stageA[pl.ds(CH + o, 16)]
            bto = s_bto[sl]
            bti = s_bti[sl]
            lbl = plsc.load_gather(s_tc, [row5, bti])
            ct = jnp.where(bto < THRESHOLD, 0.0, lbl)
            s_bto[sl] = ct
            pos = ct != 0.0
            mx = jnp.maximum(c0, c1)
            z = jnp.exp(-jnp.abs(c0 - c1))
            lse = mx + _ln(1.0 + z)
            ce = lse - jnp.where(pos, c1, c0)
            val = jnp.where(pos, 0.0, ce)
            s_val[sl] = val
            return (acc_posce + jnp.where(pos, ce, 0.0),
                    acc_npos + jnp.where(pos, 1.0, 0.0))

        return lax.fori_loop(0, GRP, p2_grp, carry, unroll=3)

    acc_posce, acc_npos = lax.fori_loop(0, NCH, p2_chunk, (fzero, fzero))
    sum_posce = vsum(acc_posce)
    npos_f = vsum(acc_npos)
    nn_f = jnp.minimum(3.0 * npos_f, float(P - 1))
    nn_f = jnp.maximum(nn_f, float(MIN_NUM_NEG))

    def count_ge(t):
        def cg(i, accs):
            o = i * 80
            new = []
            for j in range(5):
                kv = lax.bitcast_convert_type(
                    s_val[pl.ds(o + j * 16, 16)], jnp.int32)
                new.append(accs[j] + jnp.where(kv >= t, 1.0, 0.0))
            return tuple(new)

        accs = lax.fori_loop(0, P // 80, cg, (fzero,) * 5)
        return vsum(accs[0] + accs[1] + accs[2] + accs[3] + accs[4])

    def bit_iter(b, v):
        cand = v | lax.shift_left(jnp.int32(1), 30 - b)
        return jnp.where(count_ge(cand) >= nn_f, cand, v)

    vthr = lax.fori_loop(0, 31, bit_iter, jnp.int32(0))
    c_gt_f = count_ge(vthr + 1)
    t_take = nn_f - c_gt_f

    def p4_chunk(c, carry):
        pltpu.sync_copy(clss.at[wid, c], s_stageA.at[pl.ds(0, 3 * CH)])

        def p4_grp(i, carry):
            run, acc_negval, acc_r = carry
            o = i * 16
            sl = pl.ds(c * CH + o, 16)
            valv = s_val[sl]
            kv = lax.bitcast_convert_type(valv, jnp.int32)
            ct = s_bto[sl]
            pos = ct != 0.0
            eq = kv == vthr
            n_eq = _popcnt(eq)
            gt = kv > vthr

            def with_ties():
                cs = prefix_inc(jnp.where(eq, 1.0, 0.0))
                return gt | (eq & ((run + cs) <= t_take))

            taken = lax.cond(n_eq > 0, with_ties, lambda: gt)
            run = run + n_eq.astype(jnp.float32)
            acc_negval = acc_negval + jnp.where(taken, valv, 0.0)
            selm = pos | taken

            def with_ce():
                x0 = s_stageA[pl.ds(o, 16)]
                x1 = s_stageA[pl.ds(CH + o, 16)]
                x2 = s_stageA[pl.ds(2 * CH + o, 16)]
                m3 = jnp.maximum(x0, jnp.maximum(x1, x2))
                s3 = jnp.exp(x0 - m3) + jnp.exp(x1 - m3) + jnp.exp(x2 - m3)
                lse3 = m3 + _ln(s3)
                logit = jnp.where(ct == 1.0, x1, jnp.where(ct == 2.0, x2, x0))
                return acc_r + jnp.where(selm, lse3 - logit, 0.0)

            acc_r = lax.cond(_popcnt(selm) > 0, with_ce, lambda: acc_r)
            return run, acc_negval, acc_r

        return lax.fori_loop(0, GRP, p4_grp, carry, unroll=3)

    _, acc_negval, acc_r = lax.fori_loop(
        0, NCH, p4_chunk, (jnp.float32(0.0), fzero, fzero))
    sum_negval = vsum(acc_negval)
    sum_r = vsum(acc_r)

    z16 = izero

    def p5_chunk(c, carry):
        pltpu.sync_copy(lms.at[wid, c], s_stageA.at[pl.ds(0, 10 * CH)])
        pltpu.sync_copy(locs.at[wid, c], s_stageA.at[pl.ds(10 * CH, 4 * CH)])
        pltpu.sync_copy(psoa.at[c], s_stageB.at[pl.ds(0, 5 * CH)])

        def p5_grp(i, carry):
            acc_l, acc_lm, acc_nplm = carry
            o = i * 16
            sl = pl.ds(c * CH + o, 16)
            ct = s_bto[sl]
            bti = s_bti[sl]
            mlm0 = plsc.load_gather(s_lm, [bti, z16])
            pos = ct != 0.0
            pos1 = (ct == 1.0) & (mlm0 >= 0.0)

            def geom():
                px1 = s_stageB[pl.ds(o, 16)]
                py1 = s_stageB[pl.ds(CH + o, 16)]
                px2 = s_stageB[pl.ds(2 * CH + o, 16)]
                py2 = s_stageB[pl.ds(3 * CH + o, 16)]
                pcx = 0.5 * (px1 + px2)
                pcy = 0.5 * (py1 + py2)
                pw = px2 - px1
                ph = py2 - py1
                return pcx, pcy, pw, ph

            def with_loc():
                pcx, pcy, pw, ph = geom()
                tx1 = plsc.load_gather(s_tc, [z16, bti])
                ty1 = plsc.load_gather(s_tc, [z16 + 1, bti])
                tx2 = plsc.load_gather(s_tc, [z16 + 2, bti])
                ty2 = plsc.load_gather(s_tc, [z16 + 3, bti])
                gx = (0.5 * (tx1 + tx2) - pcx) / (VAR0 * pw)
                gy = (0.5 * (ty1 + ty2) - pcy) / (VAR0 * ph)
                gw = _ln((tx2 - tx1) / pw) * (1.0 / VAR1)
                gh = _ln((ty2 - ty1) / ph) * (1.0 / VAR1)
                l0 = s_stageA[pl.ds(10 * CH + o, 16)]
                l1 = s_stageA[pl.ds(11 * CH + o, 16)]
                l2 = s_stageA[pl.ds(12 * CH + o, 16)]
                l3 = s_stageA[pl.ds(13 * CH + o, 16)]
                tot = (_sl1(l0 - gx) + _sl1(l1 - gy)
                       + _sl1(l2 - gw) + _sl1(l3 - gh))
                return acc_l + jnp.where(pos, tot, 0.0)

            acc_l = lax.cond(_popcnt(pos) > 0, with_loc, lambda: acc_l)

            def with_lm():
                pcx, pcy, pw, ph = geom()
                tot = fzero
                for cc in range(10):
                    lmv = s_stageA[pl.ds(cc * CH + o, 16)]
                    mlmc = plsc.load_gather(s_lm, [bti, z16 + cc])
                    pc = pcx if cc % 2 == 0 else pcy
                    pww = pw if cc % 2 == 0 else ph
                    t = (mlmc - pc) / (VAR0 * pww)
                    tot = tot + _sl1(lmv - t)
                return (acc_lm + jnp.where(pos1, tot, 0.0),
                        acc_nplm + jnp.where(pos1, 1.0, 0.0))

            acc_lm, acc_nplm = lax.cond(_popcnt(pos1) > 0, with_lm,
                                        lambda: (acc_lm, acc_nplm))
            return acc_l, acc_lm, acc_nplm

        return lax.fori_loop(0, GRP, p5_grp, carry, unroll=3)

    acc_l, acc_lm, acc_nplm = lax.fori_loop(
        0, NCH, p5_chunk, (fzero, fzero, fzero))
    sum_l = vsum(acc_l)
    sum_lm = vsum(acc_lm)
    nplm_f = vsum(acc_nplm)

    outv = fzero
    vals = (
        sum_l,
        sum_r,
        sum_posce + sum_negval,
        sum_lm,
        npos_f,
        nn_f,
        nplm_f,
    )
    for k, sv in enumerate(vals):
        outv = jnp.where(it16 == k, sv, outv)
    s_out16[...] = outv
    pltpu.sync_copy(s_out16, out.at[wid])


def _fin_body(x_ref, o_ref):
    x = x_ref[...]
    s = jnp.sum(x, axis=0, keepdims=True)
    col = lax.broadcasted_iota(jnp.int32, (1, 16), 1)

    def pick(k):
        return jnp.sum(jnp.where(col == k, s, 0.0))

    N = jnp.maximum(pick(4), 1.0)
    N_neg = jnp.maximum(pick(5), 1.0)
    N1 = jnp.maximum(pick(6), 1.0)
    loss_l = pick(0) / N
    loss_r = pick(1) / (N + N_neg) * (NEGPOS_RATIO + 1)
    loss_c = pick(2) / (N + N_neg) * (NEGPOS_RATIO + 1)
    loss_lm = pick(3) / N1
    o = jnp.where(col == 0, loss_l, 0.0)
    o = jnp.where(col == 1, loss_r, o)
    o = jnp.where(col == 2, loss_c, o)
    o = jnp.where(col == 3, loss_lm, o)
    o_ref[...] = o


def kernel(loc_data, class_data, conf_data, landm_data, priors, targets):
    f32 = jnp.float32
    pf1 = priors[:, :2] - priors[:, 2:] / 2.0
    pf2 = priors[:, :2] + priors[:, 2:] / 2.0
    px1, py1 = pf1[:, 0], pf1[:, 1]
    px2, py2 = pf2[:, 0], pf2[:, 1]
    parea = (px2 - px1) * (py2 - py1)
    psoa = (jnp.stack([px1, py1, px2, py2, parea], 0)
            .reshape(5, NCH, CH).transpose(1, 0, 2).reshape(NCH, 5 * CH))

    def soa(x, c):
        return (x.transpose(0, 2, 1).reshape(B, c, NCH, CH)
                .transpose(0, 2, 1, 3).reshape(B, NCH, c * CH))

    confs = soa(conf_data, 2)
    clss = soa(class_data, 3)
    locs = soa(loc_data, 4)
    lms = soa(landm_data, 10)
    tr = targets[:, :, :4]
    tarea = (tr[:, :, 2] - tr[:, :, 0]) * (tr[:, :, 3] - tr[:, :, 1])
    tcomp = jnp.stack([tr[:, :, 0], tr[:, :, 1], tr[:, :, 2], tr[:, :, 3],
                       tarea, targets[:, :, 14]], 1)
    lmrow = targets[:, :, 4:14]

    mesh = plsc.VectorSubcoreMesh(core_axis_name="c", subcore_axis_name="s")
    sc = pl.kernel(
        _sc_body,
        mesh=mesh,
        compiler_params=pltpu.CompilerParams(
            needs_layout_passes=False, use_tc_tiling_on_sc=False),
        out_type=jax.ShapeDtypeStruct((B, 16), f32),
        scratch_types=[
            pltpu.VMEM((P,), f32),
            pltpu.VMEM((P,), jnp.int32),
            pltpu.VMEM((P,), f32),
            pltpu.VMEM((14 * CH,), f32),
            pltpu.VMEM((5 * CH,), f32),
            pltpu.VMEM((6, G), f32),
            pltpu.VMEM((G, 10), f32),
            pltpu.VMEM((16,), f32),
            pltpu.VMEM((16,), f32),
        ],
    )
    partials = sc(psoa, confs, clss, locs, lms, tcomp, lmrow)

    fin = pl.pallas_call(
        _fin_body,
        out_shape=jax.ShapeDtypeStruct((1, 16), f32),
    )(partials)
    return (fin[0, 0], fin[0, 1], fin[0, 2], fin[0, 3])
